# Initial kernel scaffold; baseline (speedup 1.0000x reference)
import jax
import jax.numpy as jnp
from jax.experimental import pallas as pl


def kernel(x, router_W, router_b, W1, b1, W2, b2):
    raise NotImplementedError("write your pallas kernel here")


# speedup vs baseline: 2.1306x; 2.1306x over previous
import functools
import math

import jax
import jax.numpy as jnp
from jax import lax
from jax.experimental import pallas as pl
from jax.experimental.pallas import tpu as pltpu

D_MODEL_ = 1024
D_FF_ = 2048
E_ = 8
K_ = 2
T_ = 2048


def _gelu_exact(x):
    return 0.5 * x * (1.0 + lax.erf(x * (1.0 / math.sqrt(2.0))))


def _router_body(x_ref, w_ref, b_ref, combine_ref, aux_ref):
    x = x_ref[...]
    logits = jnp.dot(x, w_ref[...], preferred_element_type=jnp.float32)
    logits = logits + b_ref[...]
    m = jnp.max(logits, axis=1, keepdims=True)
    p = jnp.exp(logits - m)
    probs = p / jnp.sum(p, axis=1, keepdims=True)

    iota = lax.broadcasted_iota(jnp.int32, probs.shape, 1)
    m1 = jnp.max(probs, axis=1, keepdims=True)
    i1 = jnp.min(jnp.where(probs == m1, iota, E_), axis=1, keepdims=True)
    probs_wo1 = jnp.where(iota == i1, -1.0, probs)
    m2 = jnp.max(probs_wo1, axis=1, keepdims=True)
    i2 = jnp.min(jnp.where(probs_wo1 == m2, iota, E_), axis=1, keepdims=True)

    den = jnp.clip(m1 + m2, 1e-9, None)
    g1 = m1 / den
    g2 = m2 / den
    onehot1 = (iota == i1).astype(jnp.float32)
    onehot2 = (iota == i2).astype(jnp.float32)
    combine_ref[...] = g1 * onehot1 + g2 * onehot2

    importance = jnp.sum(probs, axis=0) / float(T_)
    load = jnp.sum(onehot1 + onehot2, axis=0) / float(T_ * K_)
    aux = float(E_) * jnp.sum(importance * load)
    aux_ref[...] = jnp.full((1, 1), aux, dtype=jnp.float32)


def _ffn_body(x_ref, w1_ref, b1_ref, w2_ref, b2_ref, combine_ref,
              out_ref, *, bt, bf):
    e = pl.program_id(0)
    f = pl.program_id(1)
    tb = pl.program_id(2)

    x = x_ref[...]
    h = jnp.dot(x, w1_ref[0], preferred_element_type=jnp.float32) + b1_ref[0]
    h = _gelu_exact(h)
    y = jnp.dot(h, w2_ref[0], preferred_element_type=jnp.float32)

    lane = lax.broadcasted_iota(jnp.int32, (bt, E_), 1)
    col = jnp.sum(combine_ref[...] * (lane == e).astype(jnp.float32),
                  axis=1, keepdims=True)
    contrib = col * y
    contrib = contrib + jnp.where(f == 0, 1.0, 0.0) * (col * b2_ref[0])

    slot = pl.ds(tb * bt, bt)
    first = (e == 0) & (f == 0)

    @pl.when(first)
    def _():
        out_ref[slot, :] = contrib

    @pl.when(jnp.logical_not(first))
    def _():
        out_ref[slot, :] = out_ref[slot, :] + contrib


def kernel(x, router_W, router_b, W1, b1, W2, b2):
    orig_shape = x.shape
    flat = x.reshape(-1, orig_shape[-1])
    t = flat.shape[0]

    combine, aux = pl.pallas_call(
        _router_body,
        out_shape=(
            jax.ShapeDtypeStruct((t, E_), jnp.float32),
            jax.ShapeDtypeStruct((1, 1), jnp.float32),
        ),
    )(flat, router_W, router_b.reshape(1, E_))

    bt = 256
    bf = 1024
    ntb = t // bt
    nf = D_FF_ // bf
    grid = (E_, nf, ntb)

    out = pl.pallas_call(
        functools.partial(_ffn_body, bt=bt, bf=bf),
        grid=grid,
        in_specs=[
            pl.BlockSpec((bt, D_MODEL_), lambda e, f, tb: (tb, 0)),
            pl.BlockSpec((1, D_MODEL_, bf), lambda e, f, tb: (e, 0, f)),
            pl.BlockSpec((1, 1, bf), lambda e, f, tb: (e * nf + f, 0, 0)),
            pl.BlockSpec((1, bf, D_MODEL_), lambda e, f, tb: (e, f, 0)),
            pl.BlockSpec((1, 1, D_MODEL_), lambda e, f, tb: (e, 0, 0)),
            pl.BlockSpec((bt, E_), lambda e, f, tb: (tb, 0)),
        ],
        out_specs=pl.BlockSpec((t, D_MODEL_), lambda e, f, tb: (0, 0)),
        out_shape=jax.ShapeDtypeStruct((t, D_MODEL_), jnp.float32),
    )(flat, W1, b1.reshape(E_ * nf, 1, bf), W2, b2.reshape(E_, 1, D_MODEL_),
      combine)

    return out.reshape(orig_shape), aux.reshape(())


# speedup vs baseline: 2.7458x; 1.2887x over previous
import functools
import math

import jax
import jax.numpy as jnp
from jax import lax
from jax.experimental import pallas as pl
from jax.experimental.pallas import tpu as pltpu
from jax.experimental.pallas import tpu_sc as plsc

D_ = 1024
F_ = 2048
E_ = 8
K_ = 2
T_ = 2048
A_ = T_ * K_
B_ = 256
NB_ = 23
NPAD_ = NB_ * B_
BTC_ = 256

NW_ = 32
CH_ = 32


def _gelu_exact(x):
    return 0.5 * x * (1.0 + lax.erf(x * (1.0 / math.sqrt(2.0))))


def _router_meta_body(x_ref, w_ref, b_ref,
                      pos0_ref, pos1_ref, g0_ref, g1_ref, bexp_ref, aux_ref):
    x = x_ref[...]
    logits = jnp.dot(x, w_ref[...], preferred_element_type=jnp.float32)
    logits = logits + b_ref[...]
    m = jnp.max(logits, axis=1, keepdims=True)
    p = jnp.exp(logits - m)
    probs = p / jnp.sum(p, axis=1, keepdims=True)

    iota = lax.broadcasted_iota(jnp.int32, probs.shape, 1)
    m1 = jnp.max(probs, axis=1, keepdims=True)
    i1 = jnp.min(jnp.where(probs == m1, iota, E_), axis=1, keepdims=True)
    probs_wo1 = jnp.where(iota == i1, -1.0, probs)
    m2 = jnp.max(probs_wo1, axis=1, keepdims=True)
    i2 = jnp.min(jnp.where(probs_wo1 == m2, iota, E_), axis=1, keepdims=True)

    den = jnp.clip(m1 + m2, 1e-9, None)
    g0_ref[...] = m1 / den
    g1_ref[...] = m2 / den
    oh0 = (iota == i1).astype(jnp.float32)
    oh1 = (iota == i2).astype(jnp.float32)

    importance = jnp.sum(probs, axis=0) / float(T_)
    load = jnp.sum(oh0 + oh1, axis=0) / float(T_ * K_)
    aux = float(E_) * jnp.sum(importance * load)
    aux_ref[...] = jnp.full((1, 1), aux, dtype=jnp.float32)

    r = lax.broadcasted_iota(jnp.int32, (BTC_, BTC_), 0)
    c = lax.broadcasted_iota(jnp.int32, (BTC_, BTC_), 1)
    tril = (r >= c).astype(jnp.float32)

    carry = jnp.zeros((1, E_), dtype=jnp.float32)
    cums = []
    for oh in (oh0, oh1):
        blocks = []
        for bi in range(T_ // BTC_):
            blk = oh[bi * BTC_:(bi + 1) * BTC_, :]
            inc = jnp.dot(tril, blk, preferred_element_type=jnp.float32)
            inc = inc + carry
            blocks.append(inc)
            carry = inc[BTC_ - 1:BTC_, :]
        cums.append(jnp.concatenate(blocks, axis=0))
    cum0, cum1 = cums
    totals = carry

    pc = jnp.floor((totals + float(B_ - 1)) * (1.0 / B_)) * float(B_)
    er = lax.broadcasted_iota(jnp.int32, (E_, E_), 0)
    ec = lax.broadcasted_iota(jnp.int32, (E_, E_), 1)
    ustrict = (er < ec).astype(jnp.float32)
    base = jnp.dot(pc, ustrict, preferred_element_type=jnp.float32)
    ends = base + pc

    pos0 = jnp.sum(oh0 * (base + cum0 - 1.0), axis=1, keepdims=True)
    pos1 = jnp.sum(oh1 * (base + cum1 - 1.0), axis=1, keepdims=True)
    pos0_ref[...] = pos0.astype(jnp.int32)
    pos1_ref[...] = pos1.astype(jnp.int32)

    bstart = lax.broadcasted_iota(
        jnp.int32, (32, E_), 0).astype(jnp.float32) * float(B_)
    ge = (bstart >= ends).astype(jnp.float32)
    be = jnp.sum(ge, axis=1, keepdims=True)
    bexp_ref[...] = jnp.minimum(be, float(E_ - 1)).astype(jnp.int32)


def _run_router(flat, router_W, router_b):
    return pl.pallas_call(
        _router_meta_body,
        out_shape=(
            jax.ShapeDtypeStruct((T_, 1), jnp.int32),
            jax.ShapeDtypeStruct((T_, 1), jnp.int32),
            jax.ShapeDtypeStruct((T_, 1), jnp.float32),
            jax.ShapeDtypeStruct((T_, 1), jnp.float32),
            jax.ShapeDtypeStruct((32, 1), jnp.int32),
            jax.ShapeDtypeStruct((1, 1), jnp.float32),
        ),
    )(flat, router_W, router_b.reshape(1, E_))


def _dispatch_body(x_hbm, posk_hbm, gk_hbm, xs_hbm, gs_hbm,
                   idx_v, xbuf, gbuf, sem):
    cid = lax.axis_index("c")
    sid = lax.axis_index("s")
    wid = sid * 2 + cid
    k = wid // 16
    t0 = (wid % 16) * (T_ // 16)
    for ci in range(T_ // 16 // CH_):
        b = t0 + ci * CH_
        pltpu.sync_copy(posk_hbm.at[k, pl.ds(b, CH_)], idx_v.at[ci])
        pltpu.sync_copy(x_hbm.at[pl.ds(b, CH_)], xbuf)
        pltpu.sync_copy(gk_hbm.at[k, pl.ds(b, CH_)], gbuf)
        pltpu.async_copy(xbuf, xs_hbm.at[idx_v.at[ci]], sem).wait()
        pltpu.async_copy(gbuf, gs_hbm.at[idx_v.at[ci]], sem).wait()


def _run_dispatch(flat, posk, gk):
    mesh = plsc.VectorSubcoreMesh(core_axis_name="c", subcore_axis_name="s")
    return pl.kernel(
        _dispatch_body,
        out_type=(
            jax.ShapeDtypeStruct((NPAD_, D_), jnp.float32),
            jax.ShapeDtypeStruct((NPAD_,), jnp.float32),
        ),
        mesh=mesh,
        scratch_types=[
            pltpu.VMEM((T_ // 16 // CH_, CH_), jnp.int32),
            pltpu.VMEM((CH_, D_), jnp.float32),
            pltpu.VMEM((CH_,), jnp.float32),
            pltpu.SemaphoreType.DMA,
        ],
    )(flat, posk, gk)


def _ffn_body(bexp_sref, x_ref, w1_ref, b1_ref, w2_ref, b2_ref, gs_ref,
              out_ref):
    del bexp_sref
    h = jnp.dot(x_ref[...], w1_ref[0], preferred_element_type=jnp.float32)
    h = _gelu_exact(h + b1_ref[0])
    y = jnp.dot(h, w2_ref[0], preferred_element_type=jnp.float32)
    out_ref[...] = (y + b2_ref[0]) * gs_ref[...]


def _run_ffn(xs, W1, b1, W2, b2, gs, bexp):
    grid_spec = pltpu.PrefetchScalarGridSpec(
        num_scalar_prefetch=1,
        grid=(NB_,),
        in_specs=[
            pl.BlockSpec((B_, D_), lambda i, be: (i, 0)),
            pl.BlockSpec((1, D_, F_), lambda i, be: (be[i], 0, 0)),
            pl.BlockSpec((1, 1, F_), lambda i, be: (be[i], 0, 0)),
            pl.BlockSpec((1, F_, D_), lambda i, be: (be[i], 0, 0)),
            pl.BlockSpec((1, 1, D_), lambda i, be: (be[i], 0, 0)),
            pl.BlockSpec((B_, 1), lambda i, be: (i, 0)),
        ],
        out_specs=pl.BlockSpec((B_, D_), lambda i, be: (i, 0)),
    )
    return pl.pallas_call(
        _ffn_body,
        grid_spec=grid_spec,
        out_shape=jax.ShapeDtypeStruct((NPAD_, D_), jnp.float32),
    )(bexp, xs, W1, b1.reshape(E_, 1, F_), W2, b2.reshape(E_, 1, D_),
      gs.reshape(NPAD_, 1))


def _combine_body(ys_hbm, posk_hbm, out_hbm,
                  idx0, idx1, buf0, buf1, sem):
    cid = lax.axis_index("c")
    sid = lax.axis_index("s")
    wid = sid * 2 + cid
    t0 = wid * (T_ // NW_)
    for ci in range(T_ // NW_ // CH_):
        b = t0 + ci * CH_
        pltpu.sync_copy(posk_hbm.at[0, pl.ds(b, CH_)], idx0)
        pltpu.sync_copy(posk_hbm.at[1, pl.ds(b, CH_)], idx1)
        pltpu.async_copy(ys_hbm.at[idx0], buf0, sem).wait()
        pltpu.async_copy(ys_hbm.at[idx1], buf1, sem).wait()

        def body(j, carry):
            for sseg in range(D_ // 16):
                sl = pl.ds(sseg * 16, 16)
                buf0[j, sl] = buf0[j, sl] + buf1[j, sl]
            return carry

        lax.fori_loop(0, CH_, body, 0)
        pltpu.sync_copy(buf0, out_hbm.at[pl.ds(b, CH_)])


def _run_combine(ys, posk):
    mesh = plsc.VectorSubcoreMesh(core_axis_name="c", subcore_axis_name="s")
    return pl.kernel(
        _combine_body,
        out_type=jax.ShapeDtypeStruct((T_, D_), jnp.float32),
        mesh=mesh,
        scratch_types=[
            pltpu.VMEM((CH_,), jnp.int32),
            pltpu.VMEM((CH_,), jnp.int32),
            pltpu.VMEM((CH_, D_), jnp.float32),
            pltpu.VMEM((CH_, D_), jnp.float32),
            pltpu.SemaphoreType.DMA,
        ],
    )(ys, posk)


def kernel(x, router_W, router_b, W1, b1, W2, b2):
    orig_shape = x.shape
    flat = x.reshape(-1, orig_shape[-1])

    pos0, pos1, g0, g1, bexp, aux = _run_router(flat, router_W, router_b)
    posk = jnp.concatenate([pos0.reshape(1, T_), pos1.reshape(1, T_)], axis=0)
    gk = jnp.concatenate([g0.reshape(1, T_), g1.reshape(1, T_)], axis=0)

    xs, gs = _run_dispatch(flat, posk, gk)
    ys = _run_ffn(xs, W1, b1, W2, b2, gs, bexp.reshape(-1))
    out = _run_combine(ys, posk)

    return out.reshape(orig_shape), aux.reshape(())
